# baseline (speedup 1.0000x reference)
import functools

import jax
import jax.numpy as jnp
from jax import lax
from jax.experimental import pallas as pl
from jax.experimental.pallas import tpu as pltpu
from jax.experimental.pallas import tpu_sc as plsc

N = 100000
K = 64
NW = 32
BA = 64
NB = 49
APW = NB * BA
NPAD = NW * APW
SL = NPAD // 16

CUT2 = 2.5 * 2.5
R2MIN = 1e-4


@functools.partial(
    pl.kernel,
    out_type=(
        jax.ShapeDtypeStruct((2 * 3 * NPAD,), jnp.float32),
        jax.ShapeDtypeStruct((3 * NPAD,), jnp.float32),
        jax.ShapeDtypeStruct((NW * 16,), jnp.float32),
    ),
    mesh=plsc.VectorSubcoreMesh(core_axis_name="c", subcore_axis_name="s"),
    scratch_types=[
        pltpu.VMEM((K * BA,), jnp.int32),
        pltpu.VMEM((K * BA,), jnp.float32),
        pltpu.VMEM((K * BA,), jnp.float32),
        pltpu.VMEM((K * BA,), jnp.float32),
        pltpu.VMEM((K * BA,), jnp.float32),
        pltpu.VMEM((K * BA,), jnp.float32),
        pltpu.VMEM((K * BA,), jnp.float32),
        pltpu.VMEM((BA,), jnp.float32),
        pltpu.VMEM((BA,), jnp.float32),
        pltpu.VMEM((BA,), jnp.float32),
        pltpu.VMEM((BA,), jnp.int32),
        pltpu.VMEM((APW,), jnp.float32),
        pltpu.VMEM((APW,), jnp.float32),
        pltpu.VMEM((APW,), jnp.float32),
        pltpu.VMEM((SL,), jnp.float32),
        pltpu.VMEM_SHARED((NPAD,), jnp.float32),
        pltpu.VMEM_SHARED((NPAD,), jnp.float32),
        pltpu.VMEM_SHARED((NPAD,), jnp.float32),
        pltpu.SemaphoreType.DMA,
        pltpu.SemaphoreType.DMA,
        pltpu.SemaphoreType.DMA,
    ],
)
def _lj_sc(px_h, py_h, pz_h, nn_h, nm_h, o_r, o_d, o_e,
           nmv, gx, gy, gz, rfx, rfy, rfz, pxb, pyb, pzb, nnb,
           fax, fay, faz, zb, sfx, sfy, sfz, s1, s2, s3):
    c = lax.axis_index("c")
    s = lax.axis_index("s")
    w = c * 16 + s
    base = w * APW
    sl = s * SL

    zeros16 = jnp.zeros((16,), jnp.float32)

    def zi(i, carry):
        zb[pl.ds(i * 16, 16)] = zeros16
        return carry

    lax.fori_loop(0, SL // 16, zi, 0)
    pltpu.sync_copy(zb, sfx.at[pl.ds(sl, SL)])
    pltpu.sync_copy(zb, sfy.at[pl.ds(sl, SL)])
    pltpu.sync_copy(zb, sfz.at[pl.ds(sl, SL)])
    plsc.subcore_barrier()

    iot = lax.iota(jnp.int32, 16)

    def block_body(b, ecarry):
        abase = base + b * BA
        pltpu.sync_copy(nn_h.at[pl.ds(abase, BA)], nnb)
        pltpu.sync_copy(px_h.at[pl.ds(abase, BA)], pxb)
        pltpu.sync_copy(py_h.at[pl.ds(abase, BA)], pyb)
        pltpu.sync_copy(pz_h.at[pl.ds(abase, BA)], pzb)
        blk = w * NB + b
        pltpu.sync_copy(nm_h.at[pl.ds(blk * (K * BA), K * BA)], nmv)
        cp1 = pltpu.async_copy(px_h.at[nmv], gx, s1)
        cp2 = pltpu.async_copy(py_h.at[nmv], gy, s2)
        cp3 = pltpu.async_copy(pz_h.at[nmv], gz, s3)
        cp1.wait()
        cp2.wait()
        cp3.wait()

        def k_body(k, carry):
            es, fxs, fys, fzs = carry
            nes, nfx, nfy, nfz = [], [], [], []
            for a in range(BA // 16):
                off = a * 16
                ko = k * BA + off
                j = nmv[pl.ds(ko, 16)]
                gxv = gx[pl.ds(ko, 16)]
                gyv = gy[pl.ds(ko, 16)]
                gzv = gz[pl.ds(ko, 16)]
                dx = pxb[pl.ds(off, 16)] - gxv
                dy = pyb[pl.ds(off, 16)] - gyv
                dz = pzb[pl.ds(off, 16)] - gzv
                r2 = jnp.maximum(dx * dx + dy * dy + dz * dz,
                                 jnp.float32(R2MIN))
                aidx = abase + off + iot
                m = (k < nnb[pl.ds(off, 16)]) & (r2 < CUT2) & (j != aidx)
                inv2 = 1.0 / r2
                inv6 = inv2 * inv2 * inv2
                inv12 = inv6 * inv6
                e = jnp.where(m, 4.0 * (inv12 - inv6), 0.0)
                fc = jnp.where(m, 24.0 * (2.0 * inv12 - inv6) * inv2, 0.0)
                fx = fc * dx
                fy = fc * dy
                fz = fc * dz
                rfx[pl.ds(ko, 16)] = -fx
                rfy[pl.ds(ko, 16)] = -fy
                rfz[pl.ds(ko, 16)] = -fz
                nes.append(es[a] + e)
                nfx.append(fxs[a] + fx)
                nfy.append(fys[a] + fy)
                nfz.append(fzs[a] + fz)
            return tuple(nes), tuple(nfx), tuple(nfy), tuple(nfz)

        z4 = (zeros16, zeros16, zeros16, zeros16)
        es, fxs, fys, fzs = lax.fori_loop(0, K, k_body, (z4, z4, z4, z4))
        for a in range(BA // 16):
            o = b * BA + a * 16
            fax[pl.ds(o, 16)] = fxs[a]
            fay[pl.ds(o, 16)] = fys[a]
            faz[pl.ds(o, 16)] = fzs[a]
        pltpu.sync_copy(rfx, sfx.at[nmv], add=True)
        pltpu.sync_copy(rfy, sfy.at[nmv], add=True)
        pltpu.sync_copy(rfz, sfz.at[nmv], add=True)
        return (ecarry[0] + es[0] + es[1],
                ecarry[1] + es[2] + es[3])

    e0, e1 = lax.fori_loop(0, NB, block_body, (zeros16, zeros16))

    pltpu.sync_copy(fax, o_d.at[pl.ds(base, APW)])
    pltpu.sync_copy(fay, o_d.at[pl.ds(NPAD + base, APW)])
    pltpu.sync_copy(faz, o_d.at[pl.ds(2 * NPAD + base, APW)])
    zb[pl.ds(0, 16)] = e0 + e1
    pltpu.sync_copy(zb.at[pl.ds(0, 16)], o_e.at[pl.ds(w * 16, 16)])

    plsc.subcore_barrier()
    rbase = c * (3 * NPAD)
    pltpu.sync_copy(sfx.at[pl.ds(sl, SL)], o_r.at[pl.ds(rbase + sl, SL)])
    pltpu.sync_copy(sfy.at[pl.ds(sl, SL)], o_r.at[pl.ds(rbase + NPAD + sl, SL)])
    pltpu.sync_copy(sfz.at[pl.ds(sl, SL)], o_r.at[pl.ds(rbase + 2 * NPAD + sl, SL)])


def kernel(positions, neighbor_matrix, num_neighbors, batch_idx):
    pad = NPAD - N
    px = jnp.pad(positions[:, 0], (0, pad))
    py = jnp.pad(positions[:, 1], (0, pad))
    pz = jnp.pad(positions[:, 2], (0, pad))
    nn = jnp.pad(num_neighbors, (0, pad))
    nmp = jnp.pad(neighbor_matrix, ((0, pad), (0, 0)))
    nmb = nmp.reshape(NW * NB, BA, K).transpose(0, 2, 1).reshape(-1)
    o_r, o_d, o_e = _lj_sc(px, py, pz, nn, nmb)
    r = o_r.reshape(2, 3, NPAD)
    f = r[0] + r[1] + o_d.reshape(3, NPAD)
    forces = f.T[:N]
    energies = o_e.sum().reshape(1, 1)
    return energies, forces


# speedup vs baseline: 1.5482x; 1.5482x over previous
import functools

import jax
import jax.numpy as jnp
from jax import lax
from jax.experimental import pallas as pl
from jax.experimental.pallas import tpu as pltpu
from jax.experimental.pallas import tpu_sc as plsc

N = 100000
K = 64
NW = 32
BA = 64
NB = 49
APW = NB * BA
NPAD = NW * APW
SL = NPAD // 16

CUT2 = 2.5 * 2.5
R2MIN = 1e-4


@functools.partial(
    pl.kernel,
    out_type=(
        jax.ShapeDtypeStruct((2 * 3 * NPAD,), jnp.float32),
        jax.ShapeDtypeStruct((3 * NPAD,), jnp.float32),
        jax.ShapeDtypeStruct((NW * 16,), jnp.float32),
    ),
    mesh=plsc.VectorSubcoreMesh(core_axis_name="c", subcore_axis_name="s"),
    scratch_types=[
        pltpu.VMEM((K * BA,), jnp.int32),
        pltpu.VMEM((K * BA,), jnp.float32),
        pltpu.VMEM((K * BA,), jnp.float32),
        pltpu.VMEM((K * BA,), jnp.float32),
        pltpu.VMEM((K * BA,), jnp.float32),
        pltpu.VMEM((K * BA,), jnp.float32),
        pltpu.VMEM((K * BA,), jnp.float32),
        pltpu.VMEM((BA,), jnp.float32),
        pltpu.VMEM((BA,), jnp.float32),
        pltpu.VMEM((BA,), jnp.float32),
        pltpu.VMEM((BA,), jnp.int32),
        pltpu.VMEM((APW,), jnp.float32),
        pltpu.VMEM((APW,), jnp.float32),
        pltpu.VMEM((APW,), jnp.float32),
        pltpu.VMEM((SL,), jnp.float32),
        pltpu.VMEM_SHARED((NPAD,), jnp.float32),
        pltpu.VMEM_SHARED((NPAD,), jnp.float32),
        pltpu.VMEM_SHARED((NPAD,), jnp.float32),
        pltpu.VMEM_SHARED((NPAD,), jnp.float32),
        pltpu.VMEM_SHARED((NPAD,), jnp.float32),
        pltpu.VMEM_SHARED((NPAD,), jnp.float32),
        pltpu.VMEM((K * BA,), jnp.int32),
        pltpu.SemaphoreType.DMA,
        pltpu.SemaphoreType.DMA,
        pltpu.SemaphoreType.DMA,
        pltpu.SemaphoreType.DMA,
    ],
)
def _lj_sc(px_h, py_h, pz_h, nn_h, nm_h, o_r, o_d, o_e,
           nmv, gx, gy, gz, rfx, rfy, rfz, pxb, pyb, pzb, nnb,
           fax, fay, faz, zb, spx, spy, spz, sfx, sfy, sfz, nms, s1, s2, s3, s4):
    c = lax.axis_index("c")
    s = lax.axis_index("s")
    w = c * 16 + s
    base = w * APW
    sl = s * SL

    pltpu.sync_copy(px_h.at[pl.ds(sl, SL)], spx.at[pl.ds(sl, SL)])
    pltpu.sync_copy(py_h.at[pl.ds(sl, SL)], spy.at[pl.ds(sl, SL)])
    pltpu.sync_copy(pz_h.at[pl.ds(sl, SL)], spz.at[pl.ds(sl, SL)])

    zeros16 = jnp.zeros((16,), jnp.float32)

    def zi(i, carry):
        zb[pl.ds(i * 16, 16)] = zeros16
        return carry

    lax.fori_loop(0, SL // 16, zi, 0)
    pltpu.sync_copy(zb, sfx.at[pl.ds(sl, SL)])
    pltpu.sync_copy(zb, sfy.at[pl.ds(sl, SL)])
    pltpu.sync_copy(zb, sfz.at[pl.ds(sl, SL)])
    plsc.subcore_barrier()

    iot = lax.iota(jnp.int32, 16)

    def block_body(b, ecarry):
        abase = base + b * BA
        pltpu.sync_copy(nn_h.at[pl.ds(abase, BA)], nnb)
        pltpu.sync_copy(px_h.at[pl.ds(abase, BA)], pxb)
        pltpu.sync_copy(py_h.at[pl.ds(abase, BA)], pyb)
        pltpu.sync_copy(pz_h.at[pl.ds(abase, BA)], pzb)
        blk = w * NB + b
        pltpu.sync_copy(nm_h.at[pl.ds(blk * (K * BA), K * BA)], nmv)
        cp1 = pltpu.async_copy(spx.at[nmv], gx, s1)
        cp2 = pltpu.async_copy(spy.at[nmv], gy, s2)
        cp3 = pltpu.async_copy(spz.at[nmv], gz, s3)

        @pl.when(b > 0)
        def _():
            pltpu.make_async_copy(rfx, sfx.at[nms], s4).wait()
            pltpu.make_async_copy(rfy, sfy.at[nms], s4).wait()
            pltpu.make_async_copy(rfz, sfz.at[nms], s4).wait()

        cp1.wait()
        cp2.wait()
        cp3.wait()

        def k_body(k, carry):
            es, fxs, fys, fzs = carry
            nes, nfx, nfy, nfz = [], [], [], []
            for a in range(BA // 16):
                off = a * 16
                ko = k * BA + off
                j = nmv[pl.ds(ko, 16)]
                nms[pl.ds(ko, 16)] = j
                gxv = gx[pl.ds(ko, 16)]
                gyv = gy[pl.ds(ko, 16)]
                gzv = gz[pl.ds(ko, 16)]
                dx = pxb[pl.ds(off, 16)] - gxv
                dy = pyb[pl.ds(off, 16)] - gyv
                dz = pzb[pl.ds(off, 16)] - gzv
                r2 = jnp.maximum(dx * dx + dy * dy + dz * dz,
                                 jnp.float32(R2MIN))
                aidx = abase + off + iot
                m = (k < nnb[pl.ds(off, 16)]) & (r2 < CUT2) & (j != aidx)
                inv2 = 1.0 / r2
                inv6 = inv2 * inv2 * inv2
                inv12 = inv6 * inv6
                e = jnp.where(m, 4.0 * (inv12 - inv6), 0.0)
                fc = jnp.where(m, 24.0 * (2.0 * inv12 - inv6) * inv2, 0.0)
                fx = fc * dx
                fy = fc * dy
                fz = fc * dz
                rfx[pl.ds(ko, 16)] = -fx
                rfy[pl.ds(ko, 16)] = -fy
                rfz[pl.ds(ko, 16)] = -fz
                nes.append(es[a] + e)
                nfx.append(fxs[a] + fx)
                nfy.append(fys[a] + fy)
                nfz.append(fzs[a] + fz)
            return tuple(nes), tuple(nfx), tuple(nfy), tuple(nfz)

        z4 = (zeros16, zeros16, zeros16, zeros16)
        es, fxs, fys, fzs = lax.fori_loop(0, K, k_body, (z4, z4, z4, z4))
        for a in range(BA // 16):
            o = b * BA + a * 16
            fax[pl.ds(o, 16)] = fxs[a]
            fay[pl.ds(o, 16)] = fys[a]
            faz[pl.ds(o, 16)] = fzs[a]
        pltpu.async_copy(rfx, sfx.at[nms], s4, add=True)
        pltpu.async_copy(rfy, sfy.at[nms], s4, add=True)
        pltpu.async_copy(rfz, sfz.at[nms], s4, add=True)
        return (ecarry[0] + es[0] + es[1],
                ecarry[1] + es[2] + es[3])

    e0, e1 = lax.fori_loop(0, NB, block_body, (zeros16, zeros16))
    pltpu.make_async_copy(rfx, sfx.at[nms], s4).wait()
    pltpu.make_async_copy(rfy, sfy.at[nms], s4).wait()
    pltpu.make_async_copy(rfz, sfz.at[nms], s4).wait()

    pltpu.sync_copy(fax, o_d.at[pl.ds(base, APW)])
    pltpu.sync_copy(fay, o_d.at[pl.ds(NPAD + base, APW)])
    pltpu.sync_copy(faz, o_d.at[pl.ds(2 * NPAD + base, APW)])
    zb[pl.ds(0, 16)] = e0 + e1
    pltpu.sync_copy(zb.at[pl.ds(0, 16)], o_e.at[pl.ds(w * 16, 16)])

    plsc.subcore_barrier()
    rbase = c * (3 * NPAD)
    pltpu.sync_copy(sfx.at[pl.ds(sl, SL)], o_r.at[pl.ds(rbase + sl, SL)])
    pltpu.sync_copy(sfy.at[pl.ds(sl, SL)], o_r.at[pl.ds(rbase + NPAD + sl, SL)])
    pltpu.sync_copy(sfz.at[pl.ds(sl, SL)], o_r.at[pl.ds(rbase + 2 * NPAD + sl, SL)])


def kernel(positions, neighbor_matrix, num_neighbors, batch_idx):
    pad = NPAD - N
    px = jnp.pad(positions[:, 0], (0, pad))
    py = jnp.pad(positions[:, 1], (0, pad))
    pz = jnp.pad(positions[:, 2], (0, pad))
    nn = jnp.pad(num_neighbors, (0, pad))
    nmp = jnp.pad(neighbor_matrix, ((0, pad), (0, 0)))
    nmb = nmp.reshape(NW * NB, BA, K).transpose(0, 2, 1).reshape(-1)
    o_r, o_d, o_e = _lj_sc(px, py, pz, nn, nmb)
    r = o_r.reshape(2, 3, NPAD)
    f = r[0] + r[1] + o_d.reshape(3, NPAD)
    forces = f.T[:N]
    energies = o_e.sum().reshape(1, 1)
    return energies, forces


# speedup vs baseline: 2.0228x; 1.3066x over previous
import functools

import jax
import jax.numpy as jnp
from jax import lax
from jax.experimental import pallas as pl
from jax.experimental.pallas import tpu as pltpu
from jax.experimental.pallas import tpu_sc as plsc

N = 100000
K = 64
NW = 32
BA = 64
NB = 49
APW = NB * BA
NPAD = NW * APW
SL = NPAD // 16

CUT2 = 2.5 * 2.5
R2MIN = 1e-4


@functools.partial(
    pl.kernel,
    out_type=(
        jax.ShapeDtypeStruct((2 * 3 * NPAD,), jnp.float32),
        jax.ShapeDtypeStruct((3 * NPAD,), jnp.float32),
        jax.ShapeDtypeStruct((NW * 16,), jnp.float32),
    ),
    mesh=plsc.VectorSubcoreMesh(core_axis_name="c", subcore_axis_name="s"),
    scratch_types=[
        pltpu.VMEM((K * BA,), jnp.int32),
        pltpu.VMEM((K * BA,), jnp.float32),
        pltpu.VMEM((K * BA,), jnp.float32),
        pltpu.VMEM((K * BA,), jnp.float32),
        pltpu.VMEM((K * BA,), jnp.float32),
        pltpu.VMEM((K * BA,), jnp.float32),
        pltpu.VMEM((K * BA,), jnp.float32),
        pltpu.VMEM((BA,), jnp.float32),
        pltpu.VMEM((BA,), jnp.float32),
        pltpu.VMEM((BA,), jnp.float32),
        pltpu.VMEM((BA,), jnp.int32),
        pltpu.VMEM((APW,), jnp.float32),
        pltpu.VMEM((APW,), jnp.float32),
        pltpu.VMEM((APW,), jnp.float32),
        pltpu.VMEM((SL,), jnp.float32),
        pltpu.VMEM_SHARED((NPAD,), jnp.float32),
        pltpu.VMEM_SHARED((NPAD,), jnp.float32),
        pltpu.VMEM_SHARED((NPAD,), jnp.float32),
        pltpu.VMEM_SHARED((NPAD,), jnp.float32),
        pltpu.VMEM_SHARED((NPAD,), jnp.float32),
        pltpu.VMEM_SHARED((NPAD,), jnp.float32),
        pltpu.VMEM((K * BA,), jnp.int32),
        pltpu.SemaphoreType.DMA,
        pltpu.SemaphoreType.DMA,
        pltpu.SemaphoreType.DMA,
        pltpu.SemaphoreType.DMA,
    ],
)
def _lj_sc(px_h, py_h, pz_h, nn_h, nm_h, o_r, o_d, o_e,
           nmv, gx, gy, gz, rfx, rfy, rfz, pxb, pyb, pzb, nnb,
           fax, fay, faz, zb, spx, spy, spz, sfx, sfy, sfz, nms, s1, s2, s3, s4):
    c = lax.axis_index("c")
    s = lax.axis_index("s")
    w = c * 16 + s
    base = w * APW
    sl = s * SL

    pltpu.sync_copy(px_h.at[pl.ds(sl, SL)], spx.at[pl.ds(sl, SL)])
    pltpu.sync_copy(py_h.at[pl.ds(sl, SL)], spy.at[pl.ds(sl, SL)])
    pltpu.sync_copy(pz_h.at[pl.ds(sl, SL)], spz.at[pl.ds(sl, SL)])

    zeros16 = jnp.zeros((16,), jnp.float32)

    def zi(i, carry):
        zb[pl.ds(i * 16, 16)] = zeros16
        return carry

    lax.fori_loop(0, SL // 16, zi, 0)
    pltpu.sync_copy(zb, sfx.at[pl.ds(sl, SL)])
    pltpu.sync_copy(zb, sfy.at[pl.ds(sl, SL)])
    pltpu.sync_copy(zb, sfz.at[pl.ds(sl, SL)])
    plsc.subcore_barrier()

    iot = lax.iota(jnp.int32, 16)

    def block_body(b, ecarry):
        abase = base + b * BA
        pltpu.sync_copy(nn_h.at[pl.ds(abase, BA)], nnb)
        pltpu.sync_copy(px_h.at[pl.ds(abase, BA)], pxb)
        pltpu.sync_copy(py_h.at[pl.ds(abase, BA)], pyb)
        pltpu.sync_copy(pz_h.at[pl.ds(abase, BA)], pzb)
        blk = w * NB + b
        pltpu.sync_copy(nm_h.at[pl.ds(blk * (K * BA), K * BA)], nmv)
        cp1 = pltpu.async_copy(spx.at[nmv], gx, s1)
        cp2 = pltpu.async_copy(spy.at[nmv], gy, s2)
        cp3 = pltpu.async_copy(spz.at[nmv], gz, s3)

        @pl.when(b > 0)
        def _():
            pltpu.make_async_copy(rfx, sfx.at[nms], s4).wait()
            pltpu.make_async_copy(rfy, sfy.at[nms], s4).wait()
            pltpu.make_async_copy(rfz, sfz.at[nms], s4).wait()

        cp1.wait()
        cp2.wait()
        cp3.wait()

        pxv = [pxb[pl.ds(a * 16, 16)] for a in range(BA // 16)]
        pyv = [pyb[pl.ds(a * 16, 16)] for a in range(BA // 16)]
        pzv = [pzb[pl.ds(a * 16, 16)] for a in range(BA // 16)]
        nnv = [nnb[pl.ds(a * 16, 16)] for a in range(BA // 16)]
        aiv = [abase + a * 16 + iot for a in range(BA // 16)]

        def k_body(k, carry):
            es, fxs, fys, fzs = carry
            nes, nfx, nfy, nfz = [], [], [], []
            for a in range(BA // 16):
                off = a * 16
                ko = k * BA + off
                j = nmv[pl.ds(ko, 16)]
                nms[pl.ds(ko, 16)] = j
                gxv = gx[pl.ds(ko, 16)]
                gyv = gy[pl.ds(ko, 16)]
                gzv = gz[pl.ds(ko, 16)]
                dx = pxv[a] - gxv
                dy = pyv[a] - gyv
                dz = pzv[a] - gzv
                r2 = jnp.maximum(dx * dx + dy * dy + dz * dz,
                                 jnp.float32(R2MIN))
                m = (k < nnv[a]) & (r2 < CUT2) & (j != aiv[a])
                inv2 = 1.0 / r2
                inv6 = inv2 * inv2 * inv2
                inv12 = inv6 * inv6
                e = jnp.where(m, 4.0 * (inv12 - inv6), 0.0)
                fc = jnp.where(m, 24.0 * (2.0 * inv12 - inv6) * inv2, 0.0)
                fx = fc * dx
                fy = fc * dy
                fz = fc * dz
                rfx[pl.ds(ko, 16)] = -fx
                rfy[pl.ds(ko, 16)] = -fy
                rfz[pl.ds(ko, 16)] = -fz
                nes.append(es[a] + e)
                nfx.append(fxs[a] + fx)
                nfy.append(fys[a] + fy)
                nfz.append(fzs[a] + fz)
            return tuple(nes), tuple(nfx), tuple(nfy), tuple(nfz)

        z4 = (zeros16, zeros16, zeros16, zeros16)
        es, fxs, fys, fzs = lax.fori_loop(0, K, k_body, (z4, z4, z4, z4))
        for a in range(BA // 16):
            o = b * BA + a * 16
            fax[pl.ds(o, 16)] = fxs[a]
            fay[pl.ds(o, 16)] = fys[a]
            faz[pl.ds(o, 16)] = fzs[a]
        pltpu.async_copy(rfx, sfx.at[nms], s4, add=True)
        pltpu.async_copy(rfy, sfy.at[nms], s4, add=True)
        pltpu.async_copy(rfz, sfz.at[nms], s4, add=True)
        return (ecarry[0] + es[0] + es[1],
                ecarry[1] + es[2] + es[3])

    e0, e1 = lax.fori_loop(0, NB, block_body, (zeros16, zeros16))
    pltpu.make_async_copy(rfx, sfx.at[nms], s4).wait()
    pltpu.make_async_copy(rfy, sfy.at[nms], s4).wait()
    pltpu.make_async_copy(rfz, sfz.at[nms], s4).wait()

    pltpu.sync_copy(fax, o_d.at[pl.ds(base, APW)])
    pltpu.sync_copy(fay, o_d.at[pl.ds(NPAD + base, APW)])
    pltpu.sync_copy(faz, o_d.at[pl.ds(2 * NPAD + base, APW)])
    zb[pl.ds(0, 16)] = e0 + e1
    pltpu.sync_copy(zb.at[pl.ds(0, 16)], o_e.at[pl.ds(w * 16, 16)])

    plsc.subcore_barrier()
    rbase = c * (3 * NPAD)
    pltpu.sync_copy(sfx.at[pl.ds(sl, SL)], o_r.at[pl.ds(rbase + sl, SL)])
    pltpu.sync_copy(sfy.at[pl.ds(sl, SL)], o_r.at[pl.ds(rbase + NPAD + sl, SL)])
    pltpu.sync_copy(sfz.at[pl.ds(sl, SL)], o_r.at[pl.ds(rbase + 2 * NPAD + sl, SL)])


def kernel(positions, neighbor_matrix, num_neighbors, batch_idx):
    pad = NPAD - N
    px = jnp.pad(positions[:, 0], (0, pad))
    py = jnp.pad(positions[:, 1], (0, pad))
    pz = jnp.pad(positions[:, 2], (0, pad))
    nn = jnp.pad(num_neighbors, (0, pad))
    nmp = jnp.pad(neighbor_matrix, ((0, pad), (0, 0)))
    nmb = nmp.reshape(NW * NB, BA, K).transpose(0, 2, 1).reshape(-1)
    o_r, o_d, o_e = _lj_sc(px, py, pz, nn, nmb)
    r = o_r.reshape(2, 3, NPAD)
    f = r[0] + r[1] + o_d.reshape(3, NPAD)
    forces = f.T[:N]
    energies = o_e.sum().reshape(1, 1)
    return energies, forces


# speedup vs baseline: 2.0315x; 1.0043x over previous
import functools

import jax
import jax.numpy as jnp
from jax import lax
from jax.experimental import pallas as pl
from jax.experimental.pallas import tpu as pltpu
from jax.experimental.pallas import tpu_sc as plsc

N = 100000
K = 64
NW = 32
BA = 64
NB = 49
APW = NB * BA
NPAD = NW * APW
SL = NPAD // 16
KBA = K * BA
NG = BA // 16

CUT2 = 2.5 * 2.5
R2MIN = 1e-4


@functools.partial(
    pl.kernel,
    out_type=(
        jax.ShapeDtypeStruct((2 * 3 * NPAD,), jnp.float32),
        jax.ShapeDtypeStruct((3 * NPAD,), jnp.float32),
        jax.ShapeDtypeStruct((NW * 16,), jnp.float32),
    ),
    mesh=plsc.VectorSubcoreMesh(core_axis_name="c", subcore_axis_name="s"),
    scratch_types=[
        pltpu.VMEM((KBA,), jnp.int32),
        pltpu.VMEM((KBA,), jnp.int32),
        pltpu.VMEM((KBA,), jnp.float32),
        pltpu.VMEM((KBA,), jnp.float32),
        pltpu.VMEM((KBA,), jnp.float32),
        pltpu.VMEM((KBA,), jnp.float32),
        pltpu.VMEM((KBA,), jnp.float32),
        pltpu.VMEM((KBA,), jnp.float32),
        pltpu.VMEM((BA,), jnp.float32),
        pltpu.VMEM((BA,), jnp.float32),
        pltpu.VMEM((BA,), jnp.float32),
        pltpu.VMEM((BA,), jnp.int32),
        pltpu.VMEM((BA,), jnp.float32),
        pltpu.VMEM((BA,), jnp.float32),
        pltpu.VMEM((BA,), jnp.float32),
        pltpu.VMEM((BA,), jnp.int32),
        pltpu.VMEM((KBA,), jnp.float32),
        pltpu.VMEM((KBA,), jnp.float32),
        pltpu.VMEM((KBA,), jnp.float32),
        pltpu.VMEM((KBA,), jnp.int32),
        pltpu.VMEM((APW,), jnp.float32),
        pltpu.VMEM((APW,), jnp.float32),
        pltpu.VMEM((APW,), jnp.float32),
        pltpu.VMEM((SL,), jnp.float32),
        pltpu.VMEM((16,), jnp.float32),
        pltpu.VMEM_SHARED((NPAD,), jnp.float32),
        pltpu.VMEM_SHARED((NPAD,), jnp.float32),
        pltpu.VMEM_SHARED((NPAD,), jnp.float32),
        pltpu.VMEM_SHARED((NPAD,), jnp.float32),
        pltpu.VMEM_SHARED((NPAD,), jnp.float32),
        pltpu.VMEM_SHARED((NPAD,), jnp.float32),
        pltpu.SemaphoreType.DMA,
        pltpu.SemaphoreType.DMA,
        pltpu.SemaphoreType.DMA,
    ],
)
def _lj_sc(px_h, py_h, pz_h, nn_h, nm_h, o_r, o_d, o_e,
           nmvA, nmvB, gxA, gyA, gzA, gxB, gyB, gzB,
           pxbA, pybA, pzbA, nnbA, pxbB, pybB, pzbB, nnbB,
           rfx, rfy, rfz, nms, fax, fay, faz, zb, eacc,
           spx, spy, spz, sfx, sfy, sfz, sgA, sgB, ssc):
    c = lax.axis_index("c")
    s = lax.axis_index("s")
    w = c * 16 + s
    base = w * APW
    sl = s * SL

    slotA = (nmvA, gxA, gyA, gzA, pxbA, pybA, pzbA, nnbA, sgA)
    slotB = (nmvB, gxB, gyB, gzB, pxbB, pybB, pzbB, nnbB, sgB)

    pltpu.sync_copy(px_h.at[pl.ds(sl, SL)], spx.at[pl.ds(sl, SL)])
    pltpu.sync_copy(py_h.at[pl.ds(sl, SL)], spy.at[pl.ds(sl, SL)])
    pltpu.sync_copy(pz_h.at[pl.ds(sl, SL)], spz.at[pl.ds(sl, SL)])

    zeros16 = jnp.zeros((16,), jnp.float32)

    def zi(i, carry):
        zb[pl.ds(i * 16, 16)] = zeros16
        return carry

    lax.fori_loop(0, SL // 16, zi, 0)
    pltpu.sync_copy(zb, sfx.at[pl.ds(sl, SL)])
    pltpu.sync_copy(zb, sfy.at[pl.ds(sl, SL)])
    pltpu.sync_copy(zb, sfz.at[pl.ds(sl, SL)])
    plsc.subcore_barrier()

    iot = lax.iota(jnp.int32, 16)

    def fire(b, slot):
        nmv, gx, gy, gz, pxb, pyb, pzb, nnb, sg = slot
        abase = base + b * BA
        pltpu.sync_copy(nn_h.at[pl.ds(abase, BA)], nnb)
        pltpu.sync_copy(px_h.at[pl.ds(abase, BA)], pxb)
        pltpu.sync_copy(py_h.at[pl.ds(abase, BA)], pyb)
        pltpu.sync_copy(pz_h.at[pl.ds(abase, BA)], pzb)
        pltpu.sync_copy(nm_h.at[pl.ds((w * NB + b) * KBA, KBA)], nmv)
        pltpu.async_copy(spx.at[nmv], gx, sg)
        pltpu.async_copy(spy.at[nmv], gy, sg)
        pltpu.async_copy(spz.at[nmv], gz, sg)

    def wait_gathers(slot):
        nmv, gx, gy, gz, _, _, _, _, sg = slot
        pltpu.make_async_copy(spx.at[nmv], gx, sg).wait()
        pltpu.make_async_copy(spy.at[nmv], gy, sg).wait()
        pltpu.make_async_copy(spz.at[nmv], gz, sg).wait()

    def drain_scatter():
        pltpu.make_async_copy(rfx, sfx.at[nms], ssc).wait()
        pltpu.make_async_copy(rfy, sfy.at[nms], ssc).wait()
        pltpu.make_async_copy(rfz, sfz.at[nms], ssc).wait()

    def compute(b, slot):
        nmv, gx, gy, gz, pxb, pyb, pzb, nnb, _ = slot
        abase = base + b * BA
        pxv = [pxb[pl.ds(a * 16, 16)] for a in range(NG)]
        pyv = [pyb[pl.ds(a * 16, 16)] for a in range(NG)]
        pzv = [pzb[pl.ds(a * 16, 16)] for a in range(NG)]
        nnv = [nnb[pl.ds(a * 16, 16)] for a in range(NG)]
        aiv = [abase + a * 16 + iot for a in range(NG)]

        def k_body(k, carry):
            es, fxs, fys, fzs = carry
            nes, nfx, nfy, nfz = [], [], [], []
            for a in range(NG):
                ko = k * BA + a * 16
                j = nmv[pl.ds(ko, 16)]
                nms[pl.ds(ko, 16)] = j
                dx = pxv[a] - gx[pl.ds(ko, 16)]
                dy = pyv[a] - gy[pl.ds(ko, 16)]
                dz = pzv[a] - gz[pl.ds(ko, 16)]
                r2 = jnp.maximum(dx * dx + dy * dy + dz * dz,
                                 jnp.float32(R2MIN))
                m = (k < nnv[a]) & (r2 < CUT2) & (j != aiv[a])
                inv2 = 1.0 / r2
                inv6 = inv2 * inv2 * inv2
                inv12 = inv6 * inv6
                e = jnp.where(m, 4.0 * (inv12 - inv6), 0.0)
                fc = jnp.where(m, 24.0 * (2.0 * inv12 - inv6) * inv2, 0.0)
                fx = fc * dx
                fy = fc * dy
                fz = fc * dz
                rfx[pl.ds(ko, 16)] = -fx
                rfy[pl.ds(ko, 16)] = -fy
                rfz[pl.ds(ko, 16)] = -fz
                nes.append(es[a] + e)
                nfx.append(fxs[a] + fx)
                nfy.append(fys[a] + fy)
                nfz.append(fzs[a] + fz)
            return tuple(nes), tuple(nfx), tuple(nfy), tuple(nfz)

        z4 = (zeros16,) * NG
        es, fxs, fys, fzs = lax.fori_loop(0, K, k_body, (z4, z4, z4, z4))
        for a in range(NG):
            o = b * BA + a * 16
            fax[pl.ds(o, 16)] = fxs[a]
            fay[pl.ds(o, 16)] = fys[a]
            faz[pl.ds(o, 16)] = fzs[a]
        pltpu.async_copy(rfx, sfx.at[nms], ssc, add=True)
        pltpu.async_copy(rfy, sfy.at[nms], ssc, add=True)
        pltpu.async_copy(rfz, sfz.at[nms], ssc, add=True)
        eacc[...] = eacc[...] + ((es[0] + es[1]) + (es[2] + es[3]))

    eacc[...] = zeros16

    fire(0, slotA)

    def pair_body(i, carry):
        b0 = 2 * i
        b1 = b0 + 1

        @pl.when(b1 < NB)
        def _():
            fire(b1, slotB)

        @pl.when(b0 > 0)
        def _():
            drain_scatter()

        wait_gathers(slotA)
        compute(b0, slotA)

        @pl.when(b1 < NB)
        def _():
            fire(b0 + 2, slotA)
            drain_scatter()
            wait_gathers(slotB)
            compute(b1, slotB)

        return carry

    lax.fori_loop(0, (NB + 1) // 2, pair_body, 0)
    drain_scatter()

    pltpu.sync_copy(fax, o_d.at[pl.ds(base, APW)])
    pltpu.sync_copy(fay, o_d.at[pl.ds(NPAD + base, APW)])
    pltpu.sync_copy(faz, o_d.at[pl.ds(2 * NPAD + base, APW)])
    pltpu.sync_copy(eacc, o_e.at[pl.ds(w * 16, 16)])

    plsc.subcore_barrier()
    rbase = c * (3 * NPAD)
    pltpu.sync_copy(sfx.at[pl.ds(sl, SL)], o_r.at[pl.ds(rbase + sl, SL)])
    pltpu.sync_copy(sfy.at[pl.ds(sl, SL)], o_r.at[pl.ds(rbase + NPAD + sl, SL)])
    pltpu.sync_copy(sfz.at[pl.ds(sl, SL)],
                    o_r.at[pl.ds(rbase + 2 * NPAD + sl, SL)])


def kernel(positions, neighbor_matrix, num_neighbors, batch_idx):
    pad = NPAD - N
    px = jnp.pad(positions[:, 0], (0, pad))
    py = jnp.pad(positions[:, 1], (0, pad))
    pz = jnp.pad(positions[:, 2], (0, pad))
    nn = jnp.pad(num_neighbors, (0, pad))
    nmp = jnp.pad(neighbor_matrix, ((0, pad), (0, 0)))
    nmb = nmp.reshape(NW * NB, BA, K).transpose(0, 2, 1).reshape(-1)
    o_r, o_d, o_e = _lj_sc(px, py, pz, nn, nmb)
    r = o_r.reshape(2, 3, NPAD)
    f = r[0] + r[1] + o_d.reshape(3, NPAD)
    forces = f.T[:N]
    energies = o_e.sum().reshape(1, 1)
    return energies, forces


# speedup vs baseline: 2.0580x; 1.0130x over previous
import functools

import jax
import jax.numpy as jnp
from jax import lax
from jax.experimental import pallas as pl
from jax.experimental.pallas import tpu as pltpu
from jax.experimental.pallas import tpu_sc as plsc

N = 100000
K = 64
NW = 32
BA = 64
NB = 49
APW = NB * BA
NPAD = NW * APW
SL = NPAD // 16
KBA = K * BA
NG = BA // 16

CUT2 = 2.5 * 2.5
R2MIN = 1e-4


@functools.partial(
    pl.kernel,
    out_type=(
        jax.ShapeDtypeStruct((2 * 3 * NPAD,), jnp.float32),
        jax.ShapeDtypeStruct((3 * NPAD,), jnp.float32),
        jax.ShapeDtypeStruct((NW * 16,), jnp.float32),
    ),
    mesh=plsc.VectorSubcoreMesh(core_axis_name="c", subcore_axis_name="s"),
    scratch_types=[
        pltpu.VMEM((KBA,), jnp.int32),
        pltpu.VMEM((KBA,), jnp.int32),
        pltpu.VMEM((KBA,), jnp.float32),
        pltpu.VMEM((KBA,), jnp.float32),
        pltpu.VMEM((KBA,), jnp.float32),
        pltpu.VMEM((KBA,), jnp.float32),
        pltpu.VMEM((KBA,), jnp.float32),
        pltpu.VMEM((KBA,), jnp.float32),
        pltpu.VMEM((BA,), jnp.float32),
        pltpu.VMEM((BA,), jnp.float32),
        pltpu.VMEM((BA,), jnp.float32),
        pltpu.VMEM((BA,), jnp.int32),
        pltpu.VMEM((BA,), jnp.float32),
        pltpu.VMEM((BA,), jnp.float32),
        pltpu.VMEM((BA,), jnp.float32),
        pltpu.VMEM((BA,), jnp.int32),
        pltpu.VMEM((KBA,), jnp.float32),
        pltpu.VMEM((KBA,), jnp.float32),
        pltpu.VMEM((KBA,), jnp.float32),
        pltpu.VMEM((KBA,), jnp.int32),
        pltpu.VMEM((APW,), jnp.float32),
        pltpu.VMEM((APW,), jnp.float32),
        pltpu.VMEM((APW,), jnp.float32),
        pltpu.VMEM((SL,), jnp.float32),
        pltpu.VMEM((16,), jnp.float32),
        pltpu.VMEM_SHARED((NPAD,), jnp.float32),
        pltpu.VMEM_SHARED((NPAD,), jnp.float32),
        pltpu.VMEM_SHARED((NPAD,), jnp.float32),
        pltpu.VMEM_SHARED((NPAD,), jnp.float32),
        pltpu.VMEM_SHARED((NPAD,), jnp.float32),
        pltpu.VMEM_SHARED((NPAD,), jnp.float32),
        pltpu.SemaphoreType.DMA,
        pltpu.SemaphoreType.DMA,
        pltpu.SemaphoreType.DMA,
    ],
)
def _lj_sc(px_h, py_h, pz_h, nn_h, nm_h, o_r, o_d, o_e,
           nmvA, nmvB, gxA, gyA, gzA, gxB, gyB, gzB,
           pxbA, pybA, pzbA, nnbA, pxbB, pybB, pzbB, nnbB,
           rfx, rfy, rfz, nms, fax, fay, faz, zb, eacc,
           spx, spy, spz, sfx, sfy, sfz, sgA, sgB, ssc):
    c = lax.axis_index("c")
    s = lax.axis_index("s")
    w = c * 16 + s
    base = w * APW
    sl = s * SL

    slotA = (nmvA, gxA, gyA, gzA, pxbA, pybA, pzbA, nnbA, sgA)
    slotB = (nmvB, gxB, gyB, gzB, pxbB, pybB, pzbB, nnbB, sgB)

    pltpu.sync_copy(px_h.at[pl.ds(sl, SL)], spx.at[pl.ds(sl, SL)])
    pltpu.sync_copy(py_h.at[pl.ds(sl, SL)], spy.at[pl.ds(sl, SL)])
    pltpu.sync_copy(pz_h.at[pl.ds(sl, SL)], spz.at[pl.ds(sl, SL)])

    zeros16 = jnp.zeros((16,), jnp.float32)

    def zi(i, carry):
        zb[pl.ds(i * 16, 16)] = zeros16
        return carry

    lax.fori_loop(0, SL // 16, zi, 0)
    pltpu.sync_copy(zb, sfx.at[pl.ds(sl, SL)])
    pltpu.sync_copy(zb, sfy.at[pl.ds(sl, SL)])
    pltpu.sync_copy(zb, sfz.at[pl.ds(sl, SL)])
    plsc.subcore_barrier()

    iot = lax.iota(jnp.int32, 16)

    def fire(b, slot):
        nmv, gx, gy, gz, pxb, pyb, pzb, nnb, sg = slot
        abase = base + b * BA
        pltpu.sync_copy(nn_h.at[pl.ds(abase, BA)], nnb)
        pltpu.sync_copy(px_h.at[pl.ds(abase, BA)], pxb)
        pltpu.sync_copy(py_h.at[pl.ds(abase, BA)], pyb)
        pltpu.sync_copy(pz_h.at[pl.ds(abase, BA)], pzb)
        pltpu.sync_copy(nm_h.at[pl.ds((w * NB + b) * KBA, KBA)], nmv)
        pltpu.async_copy(spx.at[nmv], gx, sg)
        pltpu.async_copy(spy.at[nmv], gy, sg)
        pltpu.async_copy(spz.at[nmv], gz, sg)

    def wait_gathers(slot):
        nmv, gx, gy, gz, _, _, _, _, sg = slot
        pltpu.make_async_copy(spx.at[nmv], gx, sg).wait()
        pltpu.make_async_copy(spy.at[nmv], gy, sg).wait()
        pltpu.make_async_copy(spz.at[nmv], gz, sg).wait()

    def drain_scatter():
        pltpu.make_async_copy(rfx, sfx.at[nms], ssc).wait()
        pltpu.make_async_copy(rfy, sfy.at[nms], ssc).wait()
        pltpu.make_async_copy(rfz, sfz.at[nms], ssc).wait()

    def compute(b, slot):
        nmv, gx, gy, gz, pxb, pyb, pzb, nnb, _ = slot
        abase = base + b * BA
        pxv = [pxb[pl.ds(a * 16, 16)] for a in range(NG)]
        pyv = [pyb[pl.ds(a * 16, 16)] for a in range(NG)]
        pzv = [pzb[pl.ds(a * 16, 16)] for a in range(NG)]
        nnv = [nnb[pl.ds(a * 16, 16)] for a in range(NG)]
        aiv = [abase + a * 16 + iot for a in range(NG)]

        def k_body(k, carry):
            es, fxs, fys, fzs = carry
            nes, nfx, nfy, nfz = [], [], [], []
            for a in range(NG):
                ko = k * BA + a * 16
                j = nmv[pl.ds(ko, 16)]
                nms[pl.ds(ko, 16)] = j
                dx = pxv[a] - gx[pl.ds(ko, 16)]
                dy = pyv[a] - gy[pl.ds(ko, 16)]
                dz = pzv[a] - gz[pl.ds(ko, 16)]
                r2 = jnp.maximum(dx * dx + dy * dy + dz * dz,
                                 jnp.float32(R2MIN))
                m = (k < nnv[a]) & (r2 < CUT2) & (j != aiv[a])
                inv2 = 1.0 / r2
                u = inv2 * inv2 * inv2
                um1 = u - 1.0
                e = jnp.where(m, (4.0 * u) * um1, 0.0)
                fcn = jnp.where(m, (-24.0 * inv2) * (u * (um1 + u)), 0.0)
                rx = fcn * dx
                ry = fcn * dy
                rz = fcn * dz
                rfx[pl.ds(ko, 16)] = rx
                rfy[pl.ds(ko, 16)] = ry
                rfz[pl.ds(ko, 16)] = rz
                nes.append(es[a] + e)
                nfx.append(fxs[a] - rx)
                nfy.append(fys[a] - ry)
                nfz.append(fzs[a] - rz)
            return tuple(nes), tuple(nfx), tuple(nfy), tuple(nfz)

        z4 = (zeros16,) * NG
        es, fxs, fys, fzs = lax.fori_loop(0, K, k_body, (z4, z4, z4, z4))
        for a in range(NG):
            o = b * BA + a * 16
            fax[pl.ds(o, 16)] = fxs[a]
            fay[pl.ds(o, 16)] = fys[a]
            faz[pl.ds(o, 16)] = fzs[a]
        pltpu.async_copy(rfx, sfx.at[nms], ssc, add=True)
        pltpu.async_copy(rfy, sfy.at[nms], ssc, add=True)
        pltpu.async_copy(rfz, sfz.at[nms], ssc, add=True)
        eacc[...] = eacc[...] + ((es[0] + es[1]) + (es[2] + es[3]))

    eacc[...] = zeros16

    fire(0, slotA)

    def pair_body(i, carry):
        b0 = 2 * i
        b1 = b0 + 1

        @pl.when(b1 < NB)
        def _():
            fire(b1, slotB)

        @pl.when(b0 > 0)
        def _():
            drain_scatter()

        wait_gathers(slotA)
        compute(b0, slotA)

        @pl.when(b1 < NB)
        def _():
            fire(b0 + 2, slotA)
            drain_scatter()
            wait_gathers(slotB)
            compute(b1, slotB)

        return carry

    lax.fori_loop(0, (NB + 1) // 2, pair_body, 0)
    drain_scatter()

    pltpu.sync_copy(fax, o_d.at[pl.ds(base, APW)])
    pltpu.sync_copy(fay, o_d.at[pl.ds(NPAD + base, APW)])
    pltpu.sync_copy(faz, o_d.at[pl.ds(2 * NPAD + base, APW)])
    pltpu.sync_copy(eacc, o_e.at[pl.ds(w * 16, 16)])

    plsc.subcore_barrier()
    rbase = c * (3 * NPAD)
    pltpu.sync_copy(sfx.at[pl.ds(sl, SL)], o_r.at[pl.ds(rbase + sl, SL)])
    pltpu.sync_copy(sfy.at[pl.ds(sl, SL)], o_r.at[pl.ds(rbase + NPAD + sl, SL)])
    pltpu.sync_copy(sfz.at[pl.ds(sl, SL)],
                    o_r.at[pl.ds(rbase + 2 * NPAD + sl, SL)])


def kernel(positions, neighbor_matrix, num_neighbors, batch_idx):
    pad = NPAD - N
    px = jnp.pad(positions[:, 0], (0, pad))
    py = jnp.pad(positions[:, 1], (0, pad))
    pz = jnp.pad(positions[:, 2], (0, pad))
    nn = jnp.pad(num_neighbors, (0, pad))
    nmp = jnp.pad(neighbor_matrix, ((0, pad), (0, 0)))
    nmb = nmp.reshape(NW * NB, BA, K).transpose(0, 2, 1).reshape(-1)
    o_r, o_d, o_e = _lj_sc(px, py, pz, nn, nmb)
    r = o_r.reshape(2, 3, NPAD)
    f = r[0] + r[1] + o_d.reshape(3, NPAD)
    forces = f.T[:N]
    energies = o_e.sum().reshape(1, 1)
    return energies, forces
